# Initial kernel scaffold; baseline (speedup 1.0000x reference)
import jax
import jax.numpy as jnp
from jax.experimental import pallas as pl


def kernel(edge_index, edge_attrs, emb_weight):
    raise NotImplementedError("write your pallas kernel here")


# speedup vs baseline: 4.5521x; 4.5521x over previous
import functools

import jax
import jax.numpy as jnp
from jax import lax
from jax.experimental import pallas as pl
from jax.experimental.pallas import tpu as pltpu
from jax.experimental.pallas import tpu_sc as plsc

N_NODES = 10000
DIM = 128
N_EDGES = 320000
NUM_LAYERS = 3

NC = 2
NS = 16
NW = NC * NS
LANES = 16
CHUNK = 128
CPT = 80
EPT = CHUNK * CPT
E_PAD = NW * EPT
N_ACC = 10240
RPT = N_ACC // NS


def _sc_mesh():
    return plsc.VectorSubcoreMesh(
        core_axis_name="c", subcore_axis_name="s", num_cores=NC, num_subcores=NS
    )


_SC_PARAMS = pltpu.CompilerParams(needs_layout_passes=False)


def _hist_body(to_h, deg_out, deg_sh, ones_v, idx_v, z_v):
    cc = lax.axis_index("c")
    s = lax.axis_index("s")
    wid = cc * NS + s
    zero16 = jnp.zeros((LANES,), jnp.float32)
    one16 = jnp.ones((LANES,), jnp.float32)

    for q in range(CHUNK // LANES):
        ones_v[pl.ds(LANES * q, LANES)] = one16
        z_v[pl.ds(LANES * q, LANES)] = zero16

    base = pl.multiple_of(s * RPT, RPT)
    for b in range(RPT // CHUNK):
        pltpu.sync_copy(z_v, deg_sh.at[pl.ds(base + b * CHUNK, CHUNK)])
    plsc.subcore_barrier()

    pltpu.sync_copy(to_h.at[wid], idx_v)

    def hrow(j, _):
        pltpu.sync_copy(ones_v, deg_sh.at[idx_v.at[j]], add=True)
        return 0

    lax.fori_loop(0, CPT, hrow, 0)
    plsc.subcore_barrier()
    pltpu.sync_copy(deg_sh.at[pl.ds(base, RPT)], deg_out.at[cc, pl.ds(base, RPT)])


def _hist_call(to_p):
    return pl.kernel(
        _hist_body,
        out_type=jax.ShapeDtypeStruct((NC, N_ACC), jnp.float32),
        mesh=_sc_mesh(),
        compiler_params=_SC_PARAMS,
        scratch_types=[
            pltpu.VMEM_SHARED((N_ACC,), jnp.float32),
            pltpu.VMEM((CHUNK,), jnp.float32),
            pltpu.VMEM((CPT, CHUNK), jnp.int32),
            pltpu.VMEM((CHUNK,), jnp.float32),
        ],
    )(to_p)


def _dis_body(deg_ref, dis_ref):
    deg = deg_ref[0] + deg_ref[1]
    pos = deg > 0.0
    dis_ref[...] = jnp.where(pos, lax.rsqrt(jnp.where(pos, deg, 1.0)), 0.0)


def _dis_call(deg_partials):
    d3 = deg_partials.reshape(NC, N_ACC // DIM, DIM)
    return pl.pallas_call(
        _dis_body,
        out_shape=jax.ShapeDtypeStruct((N_ACC // DIM, DIM), jnp.float32),
    )(d3)


def _coeff_body(from_h, to_h, attr_h, dis_h, c_h, from_v, to_v, attr_v, dis_v,
                c_v):
    cc = lax.axis_index("c")
    s = lax.axis_index("s")
    wid = cc * NS + s

    pltpu.sync_copy(dis_h, dis_v)
    pltpu.sync_copy(from_h.at[wid], from_v)
    pltpu.sync_copy(to_h.at[wid], to_v)
    pltpu.sync_copy(attr_h.at[wid], attr_v)

    def crow(j, _):
        for q in range(CHUNK // LANES):
            sl = pl.ds(LANES * q, LANES)
            f16 = from_v[j, sl]
            t16 = to_v[j, sl]
            df = plsc.load_gather(
                dis_v, [lax.shift_right_logical(f16, 7), f16 & 127])
            dt = plsc.load_gather(
                dis_v, [lax.shift_right_logical(t16, 7), t16 & 127])
            a16 = attr_v[j, sl]
            c_v[j, sl] = df * dt * (1.0 + jnp.exp(-a16))
        return 0

    lax.fori_loop(0, CPT, crow, 0)
    pltpu.sync_copy(c_v, c_h.at[wid])


def _coeff_call(from_p, to_p, attr_p, dis):
    return pl.kernel(
        _coeff_body,
        out_type=jax.ShapeDtypeStruct((NW, CPT, CHUNK), jnp.float32),
        mesh=_sc_mesh(),
        compiler_params=_SC_PARAMS,
        scratch_types=[
            pltpu.VMEM((CPT, CHUNK), jnp.int32),
            pltpu.VMEM((CPT, CHUNK), jnp.int32),
            pltpu.VMEM((CPT, CHUNK), jnp.float32),
            pltpu.VMEM((N_ACC // DIM, DIM), jnp.float32),
            pltpu.VMEM((CPT, CHUNK), jnp.float32),
        ],
    )(from_p, to_p, attr_p, dis)


def _layer_body(from_h, to_h, c_h, emb_h, p_h, acc_sh, from_v, to_v, c_v, gbuf,
                gsem):
    cc = lax.axis_index("c")
    s = lax.axis_index("s")
    wid = cc * NS + s
    zero16 = jnp.zeros((LANES,), jnp.float32)

    def zrow(r, _):
        for q in range(DIM // LANES):
            gbuf[r, pl.ds(LANES * q, LANES)] = zero16
        return 0

    lax.fori_loop(0, CHUNK, zrow, 0)
    base = pl.multiple_of(s * RPT, RPT)
    for b in range(RPT // CHUNK):
        pltpu.sync_copy(gbuf, acc_sh.at[pl.ds(base + b * CHUNK, CHUNK)])

    pltpu.sync_copy(from_h.at[wid], from_v)
    pltpu.sync_copy(to_h.at[wid], to_v)
    pltpu.sync_copy(c_h.at[wid], c_v)
    plsc.subcore_barrier()

    def step(j, _):
        pltpu.async_copy(emb_h.at[from_v.at[j]], gbuf, gsem).wait()
        jidx = jnp.full((LANES,), j, jnp.int32)

        def srow(e, _):
            cb = plsc.load_gather(c_v, [jidx, jnp.full((LANES,), e, jnp.int32)])
            for q in range(DIM // LANES):
                sl = pl.ds(LANES * q, LANES)
                gbuf[e, sl] = gbuf[e, sl] * cb
            return 0

        lax.fori_loop(0, CHUNK, srow, 0)
        pltpu.sync_copy(gbuf, acc_sh.at[to_v.at[j]], add=True)
        return 0

    lax.fori_loop(0, CPT, step, 0)
    plsc.subcore_barrier()
    pltpu.sync_copy(acc_sh.at[pl.ds(base, RPT)], p_h.at[cc, pl.ds(base, RPT)])


def _layer_call(from_p, to_p, c, emb):
    return pl.kernel(
        _layer_body,
        out_type=jax.ShapeDtypeStruct((NC, N_ACC, DIM), jnp.float32),
        mesh=_sc_mesh(),
        compiler_params=_SC_PARAMS,
        scratch_types=[
            pltpu.VMEM_SHARED((N_ACC, DIM), jnp.float32),
            pltpu.VMEM((CPT, CHUNK), jnp.int32),
            pltpu.VMEM((CPT, CHUNK), jnp.int32),
            pltpu.VMEM((CPT, CHUNK), jnp.float32),
            pltpu.VMEM((CHUNK, DIM), jnp.float32),
            pltpu.SemaphoreType.DMA,
        ],
    )(from_p, to_p, c, emb)


def _combine_body(p_ref, acc_ref, emb_ref, accout_ref, *, scale):
    p = p_ref[...]
    e = p[0] + p[1]
    emb_ref[...] = e
    accout_ref[...] = (acc_ref[...] + e) * scale


_COMBINE_ROWS = 1000


def _combine_call(p, acc, scale):
    grid = N_NODES // _COMBINE_ROWS
    return pl.pallas_call(
        functools.partial(_combine_body, scale=scale),
        grid=(grid,),
        in_specs=[
            pl.BlockSpec((2, _COMBINE_ROWS, DIM), lambda i: (0, i, 0)),
            pl.BlockSpec((_COMBINE_ROWS, DIM), lambda i: (i, 0)),
        ],
        out_specs=[
            pl.BlockSpec((_COMBINE_ROWS, DIM), lambda i: (i, 0)),
            pl.BlockSpec((_COMBINE_ROWS, DIM), lambda i: (i, 0)),
        ],
        out_shape=[
            jax.ShapeDtypeStruct((N_NODES, DIM), jnp.float32),
            jax.ShapeDtypeStruct((N_NODES, DIM), jnp.float32),
        ],
    )(p, acc)


def kernel(edge_index, edge_attrs, emb_weight):
    from_ = edge_index[0].astype(jnp.int32)
    to_ = edge_index[1].astype(jnp.int32)
    attr = edge_attrs.astype(jnp.float32)

    pad = E_PAD - N_EDGES
    dummy = N_NODES + (jnp.arange(pad, dtype=jnp.int32) % (N_ACC - N_NODES))
    from_p = jnp.concatenate([from_, jnp.zeros((pad,), jnp.int32)]).reshape(
        NW, CPT, CHUNK)
    to_p = jnp.concatenate([to_, dummy]).reshape(NW, CPT, CHUNK)
    attr_p = jnp.concatenate([attr, jnp.zeros((pad,), jnp.float32)]).reshape(
        NW, CPT, CHUNK)

    deg_partials = _hist_call(to_p)
    dis = _dis_call(deg_partials)
    c = _coeff_call(from_p, to_p, attr_p, dis)

    emb = emb_weight
    acc = emb_weight
    for layer in range(NUM_LAYERS):
        partials = _layer_call(from_p, to_p, c, emb)
        scale = 1.0 / (NUM_LAYERS + 1) if layer == NUM_LAYERS - 1 else 1.0
        emb, acc = _combine_call(partials, acc, scale)

    return (emb_weight, acc)
